# baseline (speedup 1.0000x reference)
import functools

import jax
import jax.numpy as jnp
from jax import lax
from jax.experimental import pallas as pl
from jax.experimental.pallas import tpu as pltpu
from jax.experimental.pallas import tpu_sc as plsc

B = 16384
EMB = 64
D2 = 2 * EMB
NV = 1000000
NVP = NV // 2
NC = 2
NS = 16
NW = NC * NS
BPW = B // NW
CH = 256

NB = 2048


def _pack_side(a_ref, b_ref):
    x = jnp.concatenate([a_ref[...], b_ref[...]], axis=0)
    xt = x.T
    bits = lax.bitcast_convert_type(
        xt.astype(jnp.bfloat16), jnp.uint16).astype(jnp.uint32)
    pair = bits.reshape(NB // 2, 2, D2)
    return (pair[:, 1, :] << 16) | pair[:, 0, :]


def _relayout_body(gu_ref, mu_ref, gi_ref, mi_ref, u_out, i_out):
    u_out[...] = _pack_side(gu_ref, mu_ref)
    i_out[...] = _pack_side(gi_ref, mi_ref)


def _relayout(guT, muT, giT, miT):
    grid = (NV + NB - 1) // NB
    return pl.pallas_call(
        _relayout_body,
        grid=(grid,),
        in_specs=[pl.BlockSpec((EMB, NB), lambda k: (0, k))] * 4,
        out_specs=[pl.BlockSpec((NB // 2, D2), lambda k: (k, 0))] * 2,
        out_shape=[jax.ShapeDtypeStruct((NVP, D2), jnp.uint32)] * 2,
    )(guT, muT, giT, miT)


@functools.lru_cache(maxsize=None)
def _make_sc_gather():
    @functools.partial(
        pl.kernel,
        mesh=plsc.VectorSubcoreMesh(core_axis_name="c", subcore_axis_name="s"),
        out_type=(
            jax.ShapeDtypeStruct((B, D2), jnp.uint32),
            jax.ShapeDtypeStruct((B, D2), jnp.uint32),
        ),
        scratch_types=(
            pltpu.VMEM((BPW,), jnp.int32),
            pltpu.VMEM((BPW,), jnp.int32),
            pltpu.VMEM((CH, D2), jnp.uint32),
            pltpu.VMEM((CH, D2), jnp.uint32),
            pltpu.SemaphoreType.DMA,
            pltpu.SemaphoreType.DMA,
        ),
    )
    def _sc_gather(users, items, ut, it, u_out, i_out,
                   uidx, iidx, ubuf, ibuf, s0, s1):
        wid = lax.axis_index("s") * NC + lax.axis_index("c")
        base = wid * BPW

        def halve(src, dst):
            pltpu.sync_copy(src.at[pl.ds(base, BPW)], dst)

            def body(j, carry):
                sl = pl.ds(j * 16, 16)
                dst[sl] = dst[sl] >> 1
                return carry

            lax.fori_loop(0, BPW // 16, body, 0)

        halve(users, uidx)
        halve(items, iidx)
        for c in range(BPW // CH):
            off = c * CH
            cu = pltpu.async_copy(ut.at[uidx.at[pl.ds(off, CH)]], ubuf, s0)
            ci = pltpu.async_copy(it.at[iidx.at[pl.ds(off, CH)]], ibuf, s1)
            cu.wait()
            pltpu.sync_copy(ubuf, u_out.at[pl.ds(base + off, CH)])
            ci.wait()
            pltpu.sync_copy(ibuf, i_out.at[pl.ds(base + off, CH)])

    return _sc_gather


BK = 2048


def _unpack(p_ref, idx_ref):
    p = p_ref[...]
    par = (idx_ref[...] & 1).reshape(BK, 1)
    bits = jnp.where(par == 1, p & jnp.uint32(0xFFFF0000), p << 16)
    return lax.bitcast_convert_type(bits, jnp.float32)


def _mlp_body(up_ref, ip_ref, u_idx_ref, i_idx_ref, w1_ref, b1_ref, w2_ref,
              b2_ref, wp_ref, bp_ref, out_ref):
    u = _unpack(up_ref, u_idx_ref)
    i = _unpack(ip_ref, i_idx_ref)
    gmf = u[:, :EMB] * i[:, :EMB]
    x = jnp.concatenate([u[:, EMB:], i[:, EMB:]], axis=1)
    h = jnp.dot(x, w1_ref[...], preferred_element_type=jnp.float32)
    h = jnp.maximum(h + b1_ref[...], 0.0)
    h = jnp.dot(h, w2_ref[...], preferred_element_type=jnp.float32)
    h = jnp.maximum(h + b2_ref[...], 0.0)
    cat = jnp.concatenate([gmf, h], axis=1)
    pred = jnp.dot(cat, wp_ref[...], preferred_element_type=jnp.float32)
    out_ref[...] = pred[:, 0] + bp_ref[0, 0]


def kernel(users, items, gmf_user_table, gmf_item_table, mlp_user_table,
           mlp_item_table, W1, b1, W2, b2, Wp, bp):
    users = users.astype(jnp.int32)
    items = items.astype(jnp.int32)
    ut, it = _relayout(gmf_user_table.T, mlp_user_table.T,
                       gmf_item_table.T, mlp_item_table.T)
    u_rows, i_rows = _make_sc_gather()(users, items, ut, it)

    grid = B // BK
    pred = pl.pallas_call(
        _mlp_body,
        grid=(grid,),
        in_specs=[
            pl.BlockSpec((BK, D2), lambda i: (i, 0)),
            pl.BlockSpec((BK, D2), lambda i: (i, 0)),
            pl.BlockSpec((BK,), lambda i: (i,)),
            pl.BlockSpec((BK,), lambda i: (i,)),
            pl.BlockSpec((D2, 128), lambda i: (0, 0)),
            pl.BlockSpec((1, 128), lambda i: (0, 0)),
            pl.BlockSpec((128, EMB), lambda i: (0, 0)),
            pl.BlockSpec((1, EMB), lambda i: (0, 0)),
            pl.BlockSpec((D2, 1), lambda i: (0, 0)),
            pl.BlockSpec((1, 1), lambda i: (0, 0)),
        ],
        out_specs=pl.BlockSpec((BK,), lambda i: (i,)),
        out_shape=jax.ShapeDtypeStruct((B,), jnp.float32),
    )(u_rows, i_rows, users, items, W1, b1.reshape(1, 128), W2,
      b2.reshape(1, EMB), Wp, bp.reshape(1, 1))
    return pred
